# Initial kernel scaffold; baseline (speedup 1.0000x reference)
import jax
import jax.numpy as jnp
from jax.experimental import pallas as pl


def kernel(x, edge_attr, poly_val, abs_val, edge_index, params):
    raise NotImplementedError("write your pallas kernel here")


# speedup vs baseline: 3.2717x; 3.2717x over previous
import functools

import jax
import jax.numpy as jnp
from jax import lax
from jax.experimental import pallas as pl
from jax.experimental.pallas import tpu as pltpu
from jax.experimental.pallas import tpu_sc as plsc

_N = 10000
_E = 320000
_H = 128
_DE = 16
_P = 10
_ORDERS = (2, 4, 10)

_NC = 2
_NS = 16
_NW = _NC * _NS
_LANES = 16
_CHUNK = 128
_NCHUNKS = _E // _CHUNK
_NPAD = 10240
_ROWS_PER_SUB = _NPAD // _NS


def _sc_message_body(h_hbm, acc_hbm, src_hbm, dst_hbm, zeros_hbm, out_hbm,
                     src_v, dst_v, acc_v, rows_v, agg_sh, sem):
    cid = lax.axis_index("c")
    sid = lax.axis_index("s")
    wid = sid * _NC + cid

    pltpu.sync_copy(zeros_hbm,
                    agg_sh.at[pl.ds(sid * _ROWS_PER_SUB, _ROWS_PER_SUB)])
    plsc.subcore_barrier()

    def do_chunk(chunk):
        base = chunk * _CHUNK
        pltpu.sync_copy(src_hbm.at[pl.ds(base, _CHUNK)], src_v)
        pltpu.sync_copy(dst_hbm.at[pl.ds(base, _CHUNK)], dst_v)
        gather = pltpu.async_copy(h_hbm.at[src_v], rows_v, sem)
        pltpu.sync_copy(acc_hbm.at[pl.ds(base, _CHUNK)], acc_v)
        gather.wait()

        def row_body(r, carry):
            for c in range(_H // _LANES):
                sl = pl.ds(c * _LANES, _LANES)
                v = rows_v[r, sl] + acc_v[r, sl]
                rows_v[r, sl] = jnp.maximum(v, 0.0)
            return carry
        lax.fori_loop(0, _CHUNK, row_body, 0)

        pltpu.sync_copy(rows_v, agg_sh.at[dst_v], add=True)

    nfull = _NCHUNKS // _NW
    nrem = _NCHUNKS - nfull * _NW

    def loop_body(k, carry):
        do_chunk(wid + _NW * k)
        return carry
    lax.fori_loop(0, nfull, loop_body, 0)

    @pl.when(wid < nrem)
    def _():
        do_chunk(_NW * nfull + wid)

    plsc.subcore_barrier()
    pltpu.sync_copy(agg_sh.at[pl.ds(sid * _ROWS_PER_SUB, _ROWS_PER_SUB)],
                    out_hbm.at[pl.ds(cid * _NPAD + sid * _ROWS_PER_SUB,
                                     _ROWS_PER_SUB)])


_sc_message = pl.kernel(
    _sc_message_body,
    out_type=jax.ShapeDtypeStruct((_NC * _NPAD, _H), jnp.float32),
    mesh=plsc.VectorSubcoreMesh(core_axis_name="c", subcore_axis_name="s"),
    scratch_types=[
        pltpu.VMEM((_CHUNK,), jnp.int32),
        pltpu.VMEM((_CHUNK,), jnp.int32),
        pltpu.VMEM((_CHUNK, _H), jnp.float32),
        pltpu.VMEM((_CHUNK, _H), jnp.float32),
        pltpu.VMEM_SHARED((_NPAD, _H), jnp.float32),
        pltpu.SemaphoreType.DMA,
    ],
)


def _enc_body(x_ref, abs_ref, wn_ref, bn_ref, wa2_ref, wa3_ref,
              h_ref, habs2_ref, habs3_ref):
    h_ref[...] = (jnp.dot(x_ref[...], wn_ref[...],
                          preferred_element_type=jnp.float32) + bn_ref[...])
    habs2_ref[...] = jnp.dot(abs_ref[...], wa2_ref[...],
                             preferred_element_type=jnp.float32)
    habs3_ref[...] = jnp.dot(abs_ref[...], wa3_ref[...],
                             preferred_element_type=jnp.float32)


def _node_encode(x, abs_val, wn, bn, wa2, wa3):
    bn2 = bn.reshape(1, _H)
    grid = 5
    bs = _N // grid
    return pl.pallas_call(
        _enc_body,
        grid=(grid,),
        in_specs=[
            pl.BlockSpec((bs, _H), lambda i: (i, 0)),
            pl.BlockSpec((bs, _P), lambda i: (i, 0)),
            pl.BlockSpec((_H, _H), lambda i: (0, 0)),
            pl.BlockSpec((1, _H), lambda i: (0, 0)),
            pl.BlockSpec((_P, _H), lambda i: (0, 0)),
            pl.BlockSpec((_P, _H), lambda i: (0, 0)),
        ],
        out_specs=[pl.BlockSpec((bs, _H), lambda i: (i, 0))] * 3,
        out_shape=[jax.ShapeDtypeStruct((_N, _H), jnp.float32)] * 3,
    )(x, abs_val, wn, bn2, wa2, wa3)


def _acc_body(feat_ref, wcat_ref, be_ref, a1_ref, a2_ref, a3_ref):
    accs = jnp.dot(feat_ref[...], wcat_ref[...],
                   preferred_element_type=jnp.float32)
    b = be_ref[...]
    a1_ref[...] = accs[:, :_H] + b
    a2_ref[...] = accs[:, _H:2 * _H] + b
    a3_ref[...] = accs[:, 2 * _H:] + b


def _edge_accumulators(feat, wcat, b_edge):
    f = _DE + _P
    grid = 80
    bs = _E // grid
    return pl.pallas_call(
        _acc_body,
        grid=(grid,),
        in_specs=[
            pl.BlockSpec((bs, f), lambda i: (i, 0)),
            pl.BlockSpec((f, 3 * _H), lambda i: (0, 0)),
            pl.BlockSpec((1, _H), lambda i: (0, 0)),
        ],
        out_specs=[pl.BlockSpec((bs, _H), lambda i: (i, 0))] * 3,
        out_shape=[jax.ShapeDtypeStruct((_E, _H), jnp.float32)] * 3,
    )(feat, wcat, b_edge.reshape(1, _H))


def _upd_body(agg_ref, h_ref, w_ref, b_ref, habs_ref, out_ref):
    agg = agg_ref[0] + agg_ref[1]
    upd = jnp.dot(agg, w_ref[...], preferred_element_type=jnp.float32)
    out_ref[...] = (h_ref[...] + jnp.maximum(upd + b_ref[...], 0.0)
                    + habs_ref[...])


def _update_h(agg2, h, w, b, habs):
    grid = 5
    bs = _N // grid
    agg3 = agg2.reshape(_NC, _NPAD, _H)[:, :_N, :]
    return pl.pallas_call(
        _upd_body,
        grid=(grid,),
        in_specs=[
            pl.BlockSpec((_NC, bs, _H), lambda i: (0, i, 0)),
            pl.BlockSpec((bs, _H), lambda i: (i, 0)),
            pl.BlockSpec((_H, _H), lambda i: (0, 0)),
            pl.BlockSpec((1, _H), lambda i: (0, 0)),
            pl.BlockSpec((bs, _H), lambda i: (i, 0)),
        ],
        out_specs=pl.BlockSpec((bs, _H), lambda i: (i, 0)),
        out_shape=jax.ShapeDtypeStruct((_N, _H), jnp.float32),
    )(agg3, h, w, b.reshape(1, _H), habs)


def _fin_body(agg_ref, h_ref, w_ref, b_ref, wh_ref, bh_ref, out_ref):
    agg = agg_ref[0] + agg_ref[1]
    upd = jnp.dot(agg, w_ref[...], preferred_element_type=jnp.float32)
    h = h_ref[...] + jnp.maximum(upd + b_ref[...], 0.0)
    out_ref[...] = (jnp.dot(h, wh_ref[...],
                            preferred_element_type=jnp.float32) + bh_ref[...])


def _final_head(agg2, h, w, b, wh, bh):
    grid = 5
    bs = _N // grid
    agg3 = agg2.reshape(_NC, _NPAD, _H)[:, :_N, :]
    return pl.pallas_call(
        _fin_body,
        grid=(grid,),
        in_specs=[
            pl.BlockSpec((_NC, bs, _H), lambda i: (0, i, 0)),
            pl.BlockSpec((bs, _H), lambda i: (i, 0)),
            pl.BlockSpec((_H, _H), lambda i: (0, 0)),
            pl.BlockSpec((1, _H), lambda i: (0, 0)),
            pl.BlockSpec((_H, 1), lambda i: (0, 0)),
            pl.BlockSpec((1, 1), lambda i: (0, 0)),
        ],
        out_specs=pl.BlockSpec((bs, 1), lambda i: (i, 0)),
        out_shape=jax.ShapeDtypeStruct((_N, 1), jnp.float32),
    )(agg3, h, w, b.reshape(1, _H), wh, bh.reshape(1, 1))


def kernel(x, edge_attr, poly_val, abs_val, edge_index, params):
    src = edge_index[0]
    dst = edge_index[1]

    cum = jnp.zeros((_P, _H), jnp.float32)
    wcats = []
    for l, order in enumerate(_ORDERS, start=1):
        cum = cum + jnp.pad(params[f'W_rel_{l}'], ((0, _P - order), (0, 0)))
        wcats.append(jnp.concatenate([params['W_edge'], cum], axis=0))
    wcat = jnp.concatenate(wcats, axis=1)
    feat = jnp.concatenate([edge_attr, poly_val], axis=1)

    wa2 = jnp.pad(params['W_abs_2'], ((0, _P - _ORDERS[1]), (0, 0)))
    wa3 = jnp.pad(params['W_abs_3'], ((0, _P - _ORDERS[2]), (0, 0)))

    h, habs2, habs3 = _node_encode(x, abs_val, params['W_node'],
                                   params['b_node'], wa2, wa3)
    acc1, acc2, acc3 = _edge_accumulators(feat, wcat, params['b_edge'])

    zeros = jnp.zeros((_ROWS_PER_SUB, _H), jnp.float32)

    agg = _sc_message(h, acc1, src, dst, zeros)
    h = _update_h(agg, h, params['W_msg_1'], params['b_msg_1'], habs2)

    agg = _sc_message(h, acc2, src, dst, zeros)
    h = _update_h(agg, h, params['W_msg_2'], params['b_msg_2'], habs3)

    agg = _sc_message(h, acc3, src, dst, zeros)
    return _final_head(agg, h, params['W_msg_3'], params['b_msg_3'],
                       params['W_head'], params['b_head'])


# speedup vs baseline: 4.4032x; 1.3458x over previous
import jax
import jax.numpy as jnp
from jax import lax
from jax.experimental import pallas as pl
from jax.experimental.pallas import tpu as pltpu
from jax.experimental.pallas import tpu_sc as plsc

_N = 10000
_E = 320000
_H = 128
_DE = 16
_P = 10
_ORDERS = (2, 4, 10)

_NC = 2
_NS = 16
_NW = _NC * _NS
_LANES = 16
_CHUNK = 64
_NCHUNKS = _E // _CHUNK
_NPAD = 10240
_ROWS_PER_SUB = _NPAD // _NS

_CPW = 160
_CPAD = 5120


def _sc_message_body(h_hbm, acc_hbm, src_hbm, dst_hbm, zeros_hbm, out_hbm,
                     src_all, dst0, dst1, rows0, rows1, acc0, acc1, agg_sh,
                     sd0, sd1, sg0, sg1, sa0, sa1, ss0, ss1):
    cid = lax.axis_index("c")
    sid = lax.axis_index("s")
    wid = sid * _NC + cid
    cstart = wid * _CPW

    rows = (rows0, rows1)
    accb = (acc0, acc1)
    dstb = (dst0, dst1)
    sd = (sd0, sd1)
    sg = (sg0, sg1)
    sa = (sa0, sa1)
    ss = (ss0, ss1)

    pltpu.sync_copy(zeros_hbm,
                    agg_sh.at[pl.ds(sid * _ROWS_PER_SUB, _ROWS_PER_SUB)])
    pltpu.sync_copy(src_hbm.at[pl.ds(cstart * _CHUNK, _CPW * _CHUNK)], src_all)

    def live(c):
        return (cstart + c) < _NCHUNKS

    def issue(c, p):
        @pl.when((c < _CPW) & live(c))
        def _():
            g = cstart + c
            pltpu.async_copy(dst_hbm.at[pl.ds(g * _CHUNK, _CHUNK)],
                             dstb[p], sd[p])
            pltpu.async_copy(h_hbm.at[src_all.at[pl.ds(c * _CHUNK, _CHUNK)]],
                             rows[p], sg[p])
            pltpu.async_copy(acc_hbm.at[pl.ds(g * _CHUNK, _CHUNK)],
                             accb[p], sa[p])

    def drain_scatter(c, p):
        @pl.when((c >= 0) & live(c))
        def _():
            pltpu.make_async_copy(rows[p], agg_sh.at[dstb[p]], ss[p]).wait()

    issue(0, 0)
    plsc.subcore_barrier()

    def step(k, carry):
        for b in range(2):
            c = 2 * k + b
            p = b
            drain_scatter(c - 1, 1 - p)
            issue(c + 1, 1 - p)

            @pl.when(live(c))
            def _():
                g = cstart + c
                pltpu.make_async_copy(dst_hbm.at[pl.ds(g * _CHUNK, _CHUNK)],
                                      dstb[p], sd[p]).wait()
                pltpu.make_async_copy(
                    h_hbm.at[src_all.at[pl.ds(c * _CHUNK, _CHUNK)]],
                    rows[p], sg[p]).wait()
                pltpu.make_async_copy(acc_hbm.at[pl.ds(g * _CHUNK, _CHUNK)],
                                      accb[p], sa[p]).wait()

                def row_body(r, carry2):
                    for cc in range(_H // _LANES):
                        sl = pl.ds(cc * _LANES, _LANES)
                        v = rows[p][r, sl] + accb[p][r, sl]
                        rows[p][r, sl] = jnp.maximum(v, 0.0)
                    return carry2
                lax.fori_loop(0, _CHUNK, row_body, 0)

                pltpu.async_copy(rows[p], agg_sh.at[dstb[p]], ss[p], add=True)
        return carry
    lax.fori_loop(0, _CPW // 2, step, 0)
    drain_scatter(_CPW - 1, 1)

    plsc.subcore_barrier()
    pltpu.sync_copy(agg_sh.at[pl.ds(sid * _ROWS_PER_SUB, _ROWS_PER_SUB)],
                    out_hbm.at[pl.ds(cid * _NPAD + sid * _ROWS_PER_SUB,
                                     _ROWS_PER_SUB)])


_sc_message = pl.kernel(
    _sc_message_body,
    out_type=jax.ShapeDtypeStruct((_NC * _NPAD, _H), jnp.float32),
    mesh=plsc.VectorSubcoreMesh(core_axis_name="c", subcore_axis_name="s"),
    scratch_types=[
        pltpu.VMEM((_CPW * _CHUNK,), jnp.int32),
        pltpu.VMEM((_CHUNK,), jnp.int32),
        pltpu.VMEM((_CHUNK,), jnp.int32),
        pltpu.VMEM((_CHUNK, _H), jnp.float32),
        pltpu.VMEM((_CHUNK, _H), jnp.float32),
        pltpu.VMEM((_CHUNK, _H), jnp.float32),
        pltpu.VMEM((_CHUNK, _H), jnp.float32),
        pltpu.VMEM_SHARED((_NPAD, _H), jnp.float32),
        pltpu.SemaphoreType.DMA,
        pltpu.SemaphoreType.DMA,
        pltpu.SemaphoreType.DMA,
        pltpu.SemaphoreType.DMA,
        pltpu.SemaphoreType.DMA,
        pltpu.SemaphoreType.DMA,
        pltpu.SemaphoreType.DMA,
        pltpu.SemaphoreType.DMA,
    ],
)


def _enc_body(x_ref, abs_ref, wn_ref, bn_ref, wa2_ref, wa3_ref,
              h_ref, habs2_ref, habs3_ref):
    h_ref[...] = (jnp.dot(x_ref[...], wn_ref[...],
                          preferred_element_type=jnp.float32) + bn_ref[...])
    habs2_ref[...] = jnp.dot(abs_ref[...], wa2_ref[...],
                             preferred_element_type=jnp.float32)
    habs3_ref[...] = jnp.dot(abs_ref[...], wa3_ref[...],
                             preferred_element_type=jnp.float32)


def _node_encode(x, abs_val, wn, bn, wa2, wa3):
    bn2 = bn.reshape(1, _H)
    grid = 5
    bs = _N // grid
    return pl.pallas_call(
        _enc_body,
        grid=(grid,),
        in_specs=[
            pl.BlockSpec((bs, _H), lambda i: (i, 0)),
            pl.BlockSpec((bs, _P), lambda i: (i, 0)),
            pl.BlockSpec((_H, _H), lambda i: (0, 0)),
            pl.BlockSpec((1, _H), lambda i: (0, 0)),
            pl.BlockSpec((_P, _H), lambda i: (0, 0)),
            pl.BlockSpec((_P, _H), lambda i: (0, 0)),
        ],
        out_specs=[pl.BlockSpec((bs, _H), lambda i: (i, 0))] * 3,
        out_shape=[jax.ShapeDtypeStruct((_N, _H), jnp.float32)] * 3,
    )(x, abs_val, wn, bn2, wa2, wa3)


def _acc1_body(ea_ref, pv_ref, we_ref, w1_ref, be_ref, a1_ref):
    acc = (jnp.dot(ea_ref[...], we_ref[...],
                   preferred_element_type=jnp.float32) + be_ref[...])
    a1_ref[...] = acc + jnp.dot(pv_ref[...], w1_ref[...],
                                preferred_element_type=jnp.float32)


def _accn_body(a_ref, pv_ref, w_ref, o_ref):
    o_ref[...] = a_ref[...] + jnp.dot(pv_ref[...], w_ref[...],
                                      preferred_element_type=jnp.float32)


def _edge_accumulator1(edge_attr, poly_val, we, wr1, b_edge):
    grid = 80
    bs = _E // grid
    return pl.pallas_call(
        _acc1_body,
        grid=(grid,),
        in_specs=[
            pl.BlockSpec((bs, _DE), lambda i: (i, 0)),
            pl.BlockSpec((bs, _P), lambda i: (i, 0)),
            pl.BlockSpec((_DE, _H), lambda i: (0, 0)),
            pl.BlockSpec((_P, _H), lambda i: (0, 0)),
            pl.BlockSpec((1, _H), lambda i: (0, 0)),
        ],
        out_specs=pl.BlockSpec((bs, _H), lambda i: (i, 0)),
        out_shape=jax.ShapeDtypeStruct((_E, _H), jnp.float32),
    )(edge_attr, poly_val, we, wr1, b_edge.reshape(1, _H))


def _edge_accumulator_next(acc_prev, poly_val, wk):
    grid = 80
    bs = _E // grid
    return pl.pallas_call(
        _accn_body,
        grid=(grid,),
        in_specs=[
            pl.BlockSpec((bs, _H), lambda i: (i, 0)),
            pl.BlockSpec((bs, _P), lambda i: (i, 0)),
            pl.BlockSpec((_P, _H), lambda i: (0, 0)),
        ],
        out_specs=pl.BlockSpec((bs, _H), lambda i: (i, 0)),
        out_shape=jax.ShapeDtypeStruct((_E, _H), jnp.float32),
    )(acc_prev, poly_val, wk)


def _upd_body(agg_ref, h_ref, w_ref, b_ref, habs_ref, out_ref):
    agg = agg_ref[0] + agg_ref[1]
    upd = jnp.dot(agg, w_ref[...], preferred_element_type=jnp.float32)
    out_ref[...] = (h_ref[...] + jnp.maximum(upd + b_ref[...], 0.0)
                    + habs_ref[...])


def _update_h(agg2, h, w, b, habs):
    grid = 5
    bs = _N // grid
    agg3 = agg2.reshape(_NC, _NPAD, _H)
    return pl.pallas_call(
        _upd_body,
        grid=(grid,),
        in_specs=[
            pl.BlockSpec((_NC, bs, _H), lambda i: (0, i, 0)),
            pl.BlockSpec((bs, _H), lambda i: (i, 0)),
            pl.BlockSpec((_H, _H), lambda i: (0, 0)),
            pl.BlockSpec((1, _H), lambda i: (0, 0)),
            pl.BlockSpec((bs, _H), lambda i: (i, 0)),
        ],
        out_specs=pl.BlockSpec((bs, _H), lambda i: (i, 0)),
        out_shape=jax.ShapeDtypeStruct((_N, _H), jnp.float32),
    )(agg3, h, w, b.reshape(1, _H), habs)


def _fin_body(agg_ref, h_ref, w_ref, b_ref, wh_ref, bh_ref, out_ref):
    agg = agg_ref[0] + agg_ref[1]
    upd = jnp.dot(agg, w_ref[...], preferred_element_type=jnp.float32)
    h = h_ref[...] + jnp.maximum(upd + b_ref[...], 0.0)
    out_ref[...] = (jnp.dot(h, wh_ref[...],
                            preferred_element_type=jnp.float32) + bh_ref[...])


def _final_head(agg2, h, w, b, wh, bh):
    grid = 5
    bs = _N // grid
    agg3 = agg2.reshape(_NC, _NPAD, _H)
    return pl.pallas_call(
        _fin_body,
        grid=(grid,),
        in_specs=[
            pl.BlockSpec((_NC, bs, _H), lambda i: (0, i, 0)),
            pl.BlockSpec((bs, _H), lambda i: (i, 0)),
            pl.BlockSpec((_H, _H), lambda i: (0, 0)),
            pl.BlockSpec((1, _H), lambda i: (0, 0)),
            pl.BlockSpec((_H, 1), lambda i: (0, 0)),
            pl.BlockSpec((1, 1), lambda i: (0, 0)),
        ],
        out_specs=pl.BlockSpec((bs, 1), lambda i: (i, 0)),
        out_shape=jax.ShapeDtypeStruct((_N, 1), jnp.float32),
    )(agg3, h, w, b.reshape(1, _H), wh, bh.reshape(1, 1))


def kernel(x, edge_attr, poly_val, abs_val, edge_index, params):
    src = edge_index[0]
    dst = edge_index[1]
    pad = _CPAD * _CHUNK - _E
    src_p = jnp.pad(src, (0, pad))

    wrs = [jnp.pad(params[f'W_rel_{l}'], ((0, _P - order), (0, 0)))
           for l, order in enumerate(_ORDERS, start=1)]

    wa2 = jnp.pad(params['W_abs_2'], ((0, _P - _ORDERS[1]), (0, 0)))
    wa3 = jnp.pad(params['W_abs_3'], ((0, _P - _ORDERS[2]), (0, 0)))

    h, habs2, habs3 = _node_encode(x, abs_val, params['W_node'],
                                   params['b_node'], wa2, wa3)
    acc1 = _edge_accumulator1(edge_attr, poly_val, params['W_edge'],
                              wrs[0], params['b_edge'])

    zeros = jnp.zeros((_ROWS_PER_SUB, _H), jnp.float32)

    agg = _sc_message(h, acc1, src_p, dst, zeros)
    acc2 = _edge_accumulator_next(acc1, poly_val, wrs[1])
    h = _update_h(agg, h, params['W_msg_1'], params['b_msg_1'], habs2)

    agg = _sc_message(h, acc2, src_p, dst, zeros)
    acc3 = _edge_accumulator_next(acc2, poly_val, wrs[2])
    h = _update_h(agg, h, params['W_msg_2'], params['b_msg_2'], habs3)

    agg = _sc_message(h, acc3, src_p, dst, zeros)
    return _final_head(agg, h, params['W_msg_3'], params['b_msg_3'],
                       params['W_head'], params['b_head'])


# speedup vs baseline: 4.6346x; 1.0526x over previous
import jax
import jax.numpy as jnp
from jax import lax
from jax.experimental import pallas as pl
from jax.experimental.pallas import tpu as pltpu
from jax.experimental.pallas import tpu_sc as plsc

_N = 10000
_E = 320000
_H = 128
_DE = 16
_P = 10
_ORDERS = (2, 4, 10)

_NC = 2
_NS = 16
_NW = _NC * _NS
_LANES = 16
_CHUNK = 64
_NCHUNKS = _E // _CHUNK
_NPAD = 10240
_ROWS_PER_SUB = _NPAD // _NS

_CPW = 160
_CPAD = 5120


def _sc_message_body(h_hbm, acc_hbm, src_hbm, dst_hbm, zeros_hbm, out_hbm,
                     src_all, dst0, dst1, rows0, rows1, acc0, acc1, agg_sh,
                     sd0, sd1, sg0, sg1, sa0, sa1, ss0, ss1):
    cid = lax.axis_index("c")
    sid = lax.axis_index("s")
    wid = sid * _NC + cid
    cstart = wid * _CPW

    rows = (rows0, rows1)
    accb = (acc0, acc1)
    dstb = (dst0, dst1)
    sd = (sd0, sd1)
    sg = (sg0, sg1)
    sa = (sa0, sa1)
    ss = (ss0, ss1)

    pltpu.sync_copy(zeros_hbm,
                    agg_sh.at[pl.ds(sid * _ROWS_PER_SUB, _ROWS_PER_SUB)])
    pltpu.sync_copy(src_hbm.at[pl.ds(cstart * _CHUNK, _CPW * _CHUNK)], src_all)

    def live(c):
        return (cstart + c) < _NCHUNKS

    def issue(c, p):
        @pl.when((c < _CPW) & live(c))
        def _():
            g = cstart + c
            pltpu.async_copy(dst_hbm.at[pl.ds(g * _CHUNK, _CHUNK)],
                             dstb[p], sd[p])
            pltpu.async_copy(h_hbm.at[src_all.at[pl.ds(c * _CHUNK, _CHUNK)]],
                             rows[p], sg[p])
            pltpu.async_copy(acc_hbm.at[pl.ds(g * _CHUNK, _CHUNK)],
                             accb[p], sa[p])

    def drain_scatter(c, p):
        @pl.when((c >= 0) & live(c))
        def _():
            pltpu.make_async_copy(rows[p], agg_sh.at[dstb[p]], ss[p]).wait()

    issue(0, 0)
    plsc.subcore_barrier()

    def step(k, carry):
        for b in range(2):
            c = 2 * k + b
            p = b
            drain_scatter(c - 1, 1 - p)
            issue(c + 1, 1 - p)

            @pl.when(live(c))
            def _():
                g = cstart + c
                pltpu.make_async_copy(dst_hbm.at[pl.ds(g * _CHUNK, _CHUNK)],
                                      dstb[p], sd[p]).wait()
                pltpu.make_async_copy(
                    h_hbm.at[src_all.at[pl.ds(c * _CHUNK, _CHUNK)]],
                    rows[p], sg[p]).wait()
                pltpu.make_async_copy(acc_hbm.at[pl.ds(g * _CHUNK, _CHUNK)],
                                      accb[p], sa[p]).wait()

                def row_body(r, carry2):
                    for cc in range(_H // _LANES):
                        sl = pl.ds(cc * _LANES, _LANES)
                        v = rows[p][r, sl] + accb[p][r, sl]
                        rows[p][r, sl] = jnp.maximum(v, 0.0)
                    return carry2
                lax.fori_loop(0, _CHUNK, row_body, 0)

                pltpu.async_copy(rows[p], agg_sh.at[dstb[p]], ss[p], add=True)
        return carry
    lax.fori_loop(0, _CPW // 2, step, 0)
    drain_scatter(_CPW - 1, 1)

    plsc.subcore_barrier()
    pltpu.sync_copy(agg_sh.at[pl.ds(sid * _ROWS_PER_SUB, _ROWS_PER_SUB)],
                    out_hbm.at[pl.ds(cid * _NPAD + sid * _ROWS_PER_SUB,
                                     _ROWS_PER_SUB)])


_sc_message = pl.kernel(
    _sc_message_body,
    out_type=jax.ShapeDtypeStruct((_NC * _NPAD, _H), jnp.float32),
    mesh=plsc.VectorSubcoreMesh(core_axis_name="c", subcore_axis_name="s"),
    scratch_types=[
        pltpu.VMEM((_CPW * _CHUNK,), jnp.int32),
        pltpu.VMEM((_CHUNK,), jnp.int32),
        pltpu.VMEM((_CHUNK,), jnp.int32),
        pltpu.VMEM((_CHUNK, _H), jnp.float32),
        pltpu.VMEM((_CHUNK, _H), jnp.float32),
        pltpu.VMEM((_CHUNK, _H), jnp.float32),
        pltpu.VMEM((_CHUNK, _H), jnp.float32),
        pltpu.VMEM_SHARED((_NPAD, _H), jnp.float32),
        pltpu.SemaphoreType.DMA,
        pltpu.SemaphoreType.DMA,
        pltpu.SemaphoreType.DMA,
        pltpu.SemaphoreType.DMA,
        pltpu.SemaphoreType.DMA,
        pltpu.SemaphoreType.DMA,
        pltpu.SemaphoreType.DMA,
        pltpu.SemaphoreType.DMA,
    ],
)


def _enc_body(x_ref, abs_ref, wn_ref, bn_ref, wa2_ref, wa3_ref,
              h_ref, habs2_ref, habs3_ref):
    h_ref[...] = (jnp.dot(x_ref[...], wn_ref[...],
                          preferred_element_type=jnp.float32) + bn_ref[...])
    habs2_ref[...] = jnp.dot(abs_ref[...], wa2_ref[...],
                             preferred_element_type=jnp.float32)
    habs3_ref[...] = jnp.dot(abs_ref[...], wa3_ref[...],
                             preferred_element_type=jnp.float32)


def _node_encode(x, abs_val, wn, bn, wa2, wa3):
    bn2 = bn.reshape(1, _H)
    grid = 5
    bs = _N // grid
    return pl.pallas_call(
        _enc_body,
        grid=(grid,),
        in_specs=[
            pl.BlockSpec((bs, _H), lambda i: (i, 0)),
            pl.BlockSpec((bs, _P), lambda i: (i, 0)),
            pl.BlockSpec((_H, _H), lambda i: (0, 0)),
            pl.BlockSpec((1, _H), lambda i: (0, 0)),
            pl.BlockSpec((_P, _H), lambda i: (0, 0)),
            pl.BlockSpec((_P, _H), lambda i: (0, 0)),
        ],
        out_specs=[pl.BlockSpec((bs, _H), lambda i: (i, 0))] * 3,
        out_shape=[jax.ShapeDtypeStruct((_N, _H), jnp.float32)] * 3,
    )(x, abs_val, wn, bn2, wa2, wa3)


def _acc_body(ea_ref, pv_ref, we_ref, w1_ref, w2_ref, w3_ref, be_ref,
              a1_ref, a2_ref, a3_ref):
    acc = (jnp.dot(ea_ref[...], we_ref[...],
                   preferred_element_type=jnp.float32) + be_ref[...])
    acc = acc + jnp.dot(pv_ref[...], w1_ref[...],
                        preferred_element_type=jnp.float32)
    a1_ref[...] = acc
    acc = acc + jnp.dot(pv_ref[...], w2_ref[...],
                        preferred_element_type=jnp.float32)
    a2_ref[...] = acc
    a3_ref[...] = acc + jnp.dot(pv_ref[...], w3_ref[...],
                                preferred_element_type=jnp.float32)


def _edge_accumulators(edge_attr, poly_val, we, wr1, wr2, wr3, b_edge):
    grid = 80
    bs = _E // grid
    wspec = pl.BlockSpec((_P, _H), lambda i: (0, 0))
    return pl.pallas_call(
        _acc_body,
        grid=(grid,),
        in_specs=[
            pl.BlockSpec((bs, _DE), lambda i: (i, 0)),
            pl.BlockSpec((bs, _P), lambda i: (i, 0)),
            pl.BlockSpec((_DE, _H), lambda i: (0, 0)),
            wspec, wspec, wspec,
            pl.BlockSpec((1, _H), lambda i: (0, 0)),
        ],
        out_specs=[pl.BlockSpec((bs, _H), lambda i: (i, 0))] * 3,
        out_shape=[jax.ShapeDtypeStruct((_E, _H), jnp.float32)] * 3,
    )(edge_attr, poly_val, we, wr1, wr2, wr3, b_edge.reshape(1, _H))


def _upd_body(agg_ref, h_ref, w_ref, b_ref, habs_ref, out_ref):
    agg = agg_ref[0] + agg_ref[1]
    upd = jnp.dot(agg, w_ref[...], preferred_element_type=jnp.float32)
    out_ref[...] = (h_ref[...] + jnp.maximum(upd + b_ref[...], 0.0)
                    + habs_ref[...])


def _update_h(agg2, h, w, b, habs):
    grid = 5
    bs = _N // grid
    agg3 = agg2.reshape(_NC, _NPAD, _H)
    return pl.pallas_call(
        _upd_body,
        grid=(grid,),
        in_specs=[
            pl.BlockSpec((_NC, bs, _H), lambda i: (0, i, 0)),
            pl.BlockSpec((bs, _H), lambda i: (i, 0)),
            pl.BlockSpec((_H, _H), lambda i: (0, 0)),
            pl.BlockSpec((1, _H), lambda i: (0, 0)),
            pl.BlockSpec((bs, _H), lambda i: (i, 0)),
        ],
        out_specs=pl.BlockSpec((bs, _H), lambda i: (i, 0)),
        out_shape=jax.ShapeDtypeStruct((_N, _H), jnp.float32),
    )(agg3, h, w, b.reshape(1, _H), habs)


def _fin_body(agg_ref, h_ref, w_ref, b_ref, wh_ref, bh_ref, out_ref):
    agg = agg_ref[0] + agg_ref[1]
    upd = jnp.dot(agg, w_ref[...], preferred_element_type=jnp.float32)
    h = h_ref[...] + jnp.maximum(upd + b_ref[...], 0.0)
    out_ref[...] = (jnp.dot(h, wh_ref[...],
                            preferred_element_type=jnp.float32) + bh_ref[...])


def _final_head(agg2, h, w, b, wh, bh):
    grid = 5
    bs = _N // grid
    agg3 = agg2.reshape(_NC, _NPAD, _H)
    return pl.pallas_call(
        _fin_body,
        grid=(grid,),
        in_specs=[
            pl.BlockSpec((_NC, bs, _H), lambda i: (0, i, 0)),
            pl.BlockSpec((bs, _H), lambda i: (i, 0)),
            pl.BlockSpec((_H, _H), lambda i: (0, 0)),
            pl.BlockSpec((1, _H), lambda i: (0, 0)),
            pl.BlockSpec((_H, 1), lambda i: (0, 0)),
            pl.BlockSpec((1, 1), lambda i: (0, 0)),
        ],
        out_specs=pl.BlockSpec((bs, 1), lambda i: (i, 0)),
        out_shape=jax.ShapeDtypeStruct((_N, 1), jnp.float32),
    )(agg3, h, w, b.reshape(1, _H), wh, bh.reshape(1, 1))


def kernel(x, edge_attr, poly_val, abs_val, edge_index, params):
    src = edge_index[0]
    dst = edge_index[1]
    pad = _CPAD * _CHUNK - _E
    src_p = jnp.pad(src, (0, pad))

    wrs = [jnp.pad(params[f'W_rel_{l}'], ((0, _P - order), (0, 0)))
           for l, order in enumerate(_ORDERS, start=1)]

    wa2 = jnp.pad(params['W_abs_2'], ((0, _P - _ORDERS[1]), (0, 0)))
    wa3 = jnp.pad(params['W_abs_3'], ((0, _P - _ORDERS[2]), (0, 0)))

    h, habs2, habs3 = _node_encode(x, abs_val, params['W_node'],
                                   params['b_node'], wa2, wa3)
    acc1, acc2, acc3 = _edge_accumulators(
        edge_attr, poly_val, params['W_edge'], wrs[0], wrs[1], wrs[2],
        params['b_edge'])

    zeros = jnp.zeros((_ROWS_PER_SUB, _H), jnp.float32)

    agg = _sc_message(h, acc1, src_p, dst, zeros)
    h = _update_h(agg, h, params['W_msg_1'], params['b_msg_1'], habs2)

    agg = _sc_message(h, acc2, src_p, dst, zeros)
    h = _update_h(agg, h, params['W_msg_2'], params['b_msg_2'], habs3)

    agg = _sc_message(h, acc3, src_p, dst, zeros)
    return _final_head(agg, h, params['W_msg_3'], params['b_msg_3'],
                       params['W_head'], params['b_head'])


# speedup vs baseline: 4.6654x; 1.0066x over previous
import jax
import jax.numpy as jnp
from jax import lax
from jax.experimental import pallas as pl
from jax.experimental.pallas import tpu as pltpu
from jax.experimental.pallas import tpu_sc as plsc

_N = 10000
_E = 320000
_H = 128
_DE = 16
_P = 10
_ORDERS = (2, 4, 10)

_NC = 2
_NS = 16
_NW = _NC * _NS
_LANES = 16
_CHUNK = 64
_NCHUNKS = _E // _CHUNK
_NPAD = 10240
_ROWS_PER_SUB = _NPAD // _NS

_CPW = 160
_CPAD = 5120


def _sc_message_body(h_hbm, acc_hbm, src_hbm, dst_hbm, zeros_hbm, out_hbm,
                     src_all, dst0, dst1, rows0, rows1, acc0, acc1, agg_sh,
                     sd0, sd1, sg0, sg1, sa0, sa1, ss0, ss1):
    cid = lax.axis_index("c")
    sid = lax.axis_index("s")
    wid = sid * _NC + cid
    cstart = wid * _CPW

    rows = (rows0, rows1)
    accb = (acc0, acc1)
    dstb = (dst0, dst1)
    sd = (sd0, sd1)
    sg = (sg0, sg1)
    sa = (sa0, sa1)
    ss = (ss0, ss1)

    pltpu.sync_copy(zeros_hbm,
                    agg_sh.at[pl.ds(sid * _ROWS_PER_SUB, _ROWS_PER_SUB)])
    pltpu.sync_copy(src_hbm.at[pl.ds(cstart * _CHUNK, _CPW * _CHUNK)], src_all)

    def live(c):
        return (cstart + c) < _NCHUNKS

    def issue(c, p):
        @pl.when((c < _CPW) & live(c))
        def _():
            g = cstart + c
            pltpu.async_copy(dst_hbm.at[pl.ds(g * _CHUNK, _CHUNK)],
                             dstb[p], sd[p])
            pltpu.async_copy(h_hbm.at[src_all.at[pl.ds(c * _CHUNK, _CHUNK)]],
                             rows[p], sg[p])
            pltpu.async_copy(acc_hbm.at[pl.ds(g * _CHUNK, _CHUNK)],
                             accb[p], sa[p])

    def drain_scatter(c, p):
        @pl.when((c >= 0) & live(c))
        def _():
            pltpu.make_async_copy(rows[p], agg_sh.at[dstb[p]], ss[p]).wait()

    issue(0, 0)
    plsc.subcore_barrier()

    def step(k, carry):
        for b in range(2):
            c = 2 * k + b
            p = b
            drain_scatter(c - 1, 1 - p)
            issue(c + 1, 1 - p)

            @pl.when(live(c))
            def _():
                g = cstart + c
                pltpu.make_async_copy(dst_hbm.at[pl.ds(g * _CHUNK, _CHUNK)],
                                      dstb[p], sd[p]).wait()
                pltpu.make_async_copy(
                    h_hbm.at[src_all.at[pl.ds(c * _CHUNK, _CHUNK)]],
                    rows[p], sg[p]).wait()
                pltpu.make_async_copy(acc_hbm.at[pl.ds(g * _CHUNK, _CHUNK)],
                                      accb[p], sa[p]).wait()

                def row_body(r, carry2):
                    for cc in range(_H // _LANES):
                        sl = pl.ds(cc * _LANES, _LANES)
                        v = rows[p][r, sl] + accb[p][r, sl]
                        rows[p][r, sl] = jnp.maximum(v, 0.0)
                    return carry2
                lax.fori_loop(0, _CHUNK, row_body, 0)

                pltpu.async_copy(rows[p], agg_sh.at[dstb[p]], ss[p], add=True)
        return carry
    lax.fori_loop(0, _CPW // 2, step, 0)
    drain_scatter(_CPW - 1, 1)

    plsc.subcore_barrier()
    pltpu.sync_copy(agg_sh.at[pl.ds(sid * _ROWS_PER_SUB, _ROWS_PER_SUB)],
                    out_hbm.at[pl.ds(cid * _NPAD + sid * _ROWS_PER_SUB,
                                     _ROWS_PER_SUB)])


_sc_message = pl.kernel(
    _sc_message_body,
    out_type=jax.ShapeDtypeStruct((_NC * _NPAD, _H), jnp.float32),
    mesh=plsc.VectorSubcoreMesh(core_axis_name="c", subcore_axis_name="s"),
    scratch_types=[
        pltpu.VMEM((_CPW * _CHUNK,), jnp.int32),
        pltpu.VMEM((_CHUNK,), jnp.int32),
        pltpu.VMEM((_CHUNK,), jnp.int32),
        pltpu.VMEM((_CHUNK, _H), jnp.float32),
        pltpu.VMEM((_CHUNK, _H), jnp.float32),
        pltpu.VMEM((_CHUNK, _H), jnp.float32),
        pltpu.VMEM((_CHUNK, _H), jnp.float32),
        pltpu.VMEM_SHARED((_NPAD, _H), jnp.float32),
        pltpu.SemaphoreType.DMA,
        pltpu.SemaphoreType.DMA,
        pltpu.SemaphoreType.DMA,
        pltpu.SemaphoreType.DMA,
        pltpu.SemaphoreType.DMA,
        pltpu.SemaphoreType.DMA,
        pltpu.SemaphoreType.DMA,
        pltpu.SemaphoreType.DMA,
    ],
)


def _enc_body(x_ref, abs_ref, wn_ref, bn_ref, wa2_ref, wa3_ref,
              h_ref, habs2_ref, habs3_ref):
    h_ref[...] = (jnp.dot(x_ref[...], wn_ref[...],
                          preferred_element_type=jnp.float32) + bn_ref[...])
    habs2_ref[...] = jnp.dot(abs_ref[...], wa2_ref[...],
                             preferred_element_type=jnp.float32)
    habs3_ref[...] = jnp.dot(abs_ref[...], wa3_ref[...],
                             preferred_element_type=jnp.float32)


def _node_encode(x, abs_val, wn, bn, wa2, wa3):
    bn2 = bn.reshape(1, _H)
    grid = 5
    bs = _N // grid
    return pl.pallas_call(
        _enc_body,
        grid=(grid,),
        in_specs=[
            pl.BlockSpec((bs, _H), lambda i: (i, 0)),
            pl.BlockSpec((bs, _P), lambda i: (i, 0)),
            pl.BlockSpec((_H, _H), lambda i: (0, 0)),
            pl.BlockSpec((1, _H), lambda i: (0, 0)),
            pl.BlockSpec((_P, _H), lambda i: (0, 0)),
            pl.BlockSpec((_P, _H), lambda i: (0, 0)),
        ],
        out_specs=[pl.BlockSpec((bs, _H), lambda i: (i, 0))] * 3,
        out_shape=[jax.ShapeDtypeStruct((_N, _H), jnp.float32)] * 3,
    )(x, abs_val, wn, bn2, wa2, wa3)


def _acc_body(ea_ref, pv_ref, we_ref, w1_ref, w2_ref, w3_ref, be_ref,
              a1_ref, a2_ref, a3_ref):
    acc = (jnp.dot(ea_ref[...], we_ref[...],
                   preferred_element_type=jnp.float32) + be_ref[...])
    acc = acc + jnp.dot(pv_ref[...], w1_ref[...],
                        preferred_element_type=jnp.float32)
    a1_ref[...] = acc
    acc = acc + jnp.dot(pv_ref[...], w2_ref[...],
                        preferred_element_type=jnp.float32)
    a2_ref[...] = acc
    a3_ref[...] = acc + jnp.dot(pv_ref[...], w3_ref[...],
                                preferred_element_type=jnp.float32)


def _edge_accumulators(edge_attr, poly_val, we, wr1, wr2, wr3, b_edge):
    grid = 40
    bs = _E // grid
    wspec = pl.BlockSpec((_P, _H), lambda i: (0, 0))
    return pl.pallas_call(
        _acc_body,
        grid=(grid,),
        in_specs=[
            pl.BlockSpec((bs, _DE), lambda i: (i, 0)),
            pl.BlockSpec((bs, _P), lambda i: (i, 0)),
            pl.BlockSpec((_DE, _H), lambda i: (0, 0)),
            wspec, wspec, wspec,
            pl.BlockSpec((1, _H), lambda i: (0, 0)),
        ],
        out_specs=[pl.BlockSpec((bs, _H), lambda i: (i, 0))] * 3,
        out_shape=[jax.ShapeDtypeStruct((_E, _H), jnp.float32)] * 3,
    )(edge_attr, poly_val, we, wr1, wr2, wr3, b_edge.reshape(1, _H))


def _upd_body(agg_ref, h_ref, w_ref, b_ref, habs_ref, out_ref):
    agg = agg_ref[0] + agg_ref[1]
    upd = jnp.dot(agg, w_ref[...], preferred_element_type=jnp.float32)
    out_ref[...] = (h_ref[...] + jnp.maximum(upd + b_ref[...], 0.0)
                    + habs_ref[...])


def _update_h(agg2, h, w, b, habs):
    grid = 5
    bs = _N // grid
    agg3 = agg2.reshape(_NC, _NPAD, _H)
    return pl.pallas_call(
        _upd_body,
        grid=(grid,),
        in_specs=[
            pl.BlockSpec((_NC, bs, _H), lambda i: (0, i, 0)),
            pl.BlockSpec((bs, _H), lambda i: (i, 0)),
            pl.BlockSpec((_H, _H), lambda i: (0, 0)),
            pl.BlockSpec((1, _H), lambda i: (0, 0)),
            pl.BlockSpec((bs, _H), lambda i: (i, 0)),
        ],
        out_specs=pl.BlockSpec((bs, _H), lambda i: (i, 0)),
        out_shape=jax.ShapeDtypeStruct((_N, _H), jnp.float32),
    )(agg3, h, w, b.reshape(1, _H), habs)


def _fin_body(agg_ref, h_ref, w_ref, b_ref, wh_ref, bh_ref, out_ref):
    agg = agg_ref[0] + agg_ref[1]
    upd = jnp.dot(agg, w_ref[...], preferred_element_type=jnp.float32)
    h = h_ref[...] + jnp.maximum(upd + b_ref[...], 0.0)
    out_ref[...] = (jnp.dot(h, wh_ref[...],
                            preferred_element_type=jnp.float32) + bh_ref[...])


def _final_head(agg2, h, w, b, wh, bh):
    grid = 5
    bs = _N // grid
    agg3 = agg2.reshape(_NC, _NPAD, _H)
    return pl.pallas_call(
        _fin_body,
        grid=(grid,),
        in_specs=[
            pl.BlockSpec((_NC, bs, _H), lambda i: (0, i, 0)),
            pl.BlockSpec((bs, _H), lambda i: (i, 0)),
            pl.BlockSpec((_H, _H), lambda i: (0, 0)),
            pl.BlockSpec((1, _H), lambda i: (0, 0)),
            pl.BlockSpec((_H, 1), lambda i: (0, 0)),
            pl.BlockSpec((1, 1), lambda i: (0, 0)),
        ],
        out_specs=pl.BlockSpec((bs, 1), lambda i: (i, 0)),
        out_shape=jax.ShapeDtypeStruct((_N, 1), jnp.float32),
    )(agg3, h, w, b.reshape(1, _H), wh, bh.reshape(1, 1))


def kernel(x, edge_attr, poly_val, abs_val, edge_index, params):
    src = edge_index[0]
    dst = edge_index[1]
    pad = _CPAD * _CHUNK - _E
    src_p = jnp.pad(src, (0, pad))

    wrs = [jnp.pad(params[f'W_rel_{l}'], ((0, _P - order), (0, 0)))
           for l, order in enumerate(_ORDERS, start=1)]

    wa2 = jnp.pad(params['W_abs_2'], ((0, _P - _ORDERS[1]), (0, 0)))
    wa3 = jnp.pad(params['W_abs_3'], ((0, _P - _ORDERS[2]), (0, 0)))

    h, habs2, habs3 = _node_encode(x, abs_val, params['W_node'],
                                   params['b_node'], wa2, wa3)
    acc1, acc2, acc3 = _edge_accumulators(
        edge_attr, poly_val, params['W_edge'], wrs[0], wrs[1], wrs[2],
        params['b_edge'])

    zeros = jnp.zeros((_ROWS_PER_SUB, _H), jnp.float32)

    agg = _sc_message(h, acc1, src_p, dst, zeros)
    h = _update_h(agg, h, params['W_msg_1'], params['b_msg_1'], habs2)

    agg = _sc_message(h, acc2, src_p, dst, zeros)
    h = _update_h(agg, h, params['W_msg_2'], params['b_msg_2'], habs3)

    agg = _sc_message(h, acc3, src_p, dst, zeros)
    return _final_head(agg, h, params['W_msg_3'], params['b_msg_3'],
                       params['W_head'], params['b_head'])
